# baseline (speedup 1.0000x reference)
import functools

import jax
import jax.numpy as jnp
from jax.experimental import pallas as pl
import jax.experimental.pallas.tpu as pltpu

N = 8192
D = 512
K = 32
R = 128
C = 128
NC = N // C
NUM_BLOCKS = N // R
NCACHE = 4

NEG = float("-inf")
BIG = N


def _normalize_body(x_ref, o_ref):
    x = x_ref[...]
    nrm = jnp.sqrt(jnp.sum(x * x, axis=1, keepdims=True))
    o_ref[...] = x / (nrm + 1e-10)


def _topk_body(b_ref, vals_ref, idx_ref, s3_ref, vc_ref, cc_ref,
               b_vmem_ref, sem):
    i = pl.program_id(0)
    j = pl.program_id(1)

    @pl.when((i == 0) & (j == 0))
    def _():
        pltpu.make_async_copy(b_ref, b_vmem_ref, sem).start()
        pltpu.make_async_copy(b_ref, b_vmem_ref, sem).wait()

    @pl.when(j == 0)
    def _():
        a = b_vmem_ref[pl.ds(i * R, R), :]
        HW = N // 4
        HC = HW // C
        for h in range(4):
            bh = b_vmem_ref[pl.ds(h * HW, HW), :]
            sim_h = jax.lax.dot_general(
                a, bh, (((1,), (1,)), ((), ())),
                preferred_element_type=jnp.float32)
            colh = jax.lax.broadcasted_iota(jnp.int32, (R, HW), 1) + h * HW
            rowh = jax.lax.broadcasted_iota(jnp.int32, (R, HW), 0) + i * R
            sim_h = jnp.where(colh == rowh, 0.0, sim_h)
            s3_ref[:, h * HC:(h + 1) * HC, :] = sim_h.reshape(R, HC, C)
        lane3 = jax.lax.broadcasted_iota(jnp.int32, (R, NC, C), 2)
        giota2 = jax.lax.broadcasted_iota(jnp.int32, (R, NC), 1)
        s3 = s3_ref[...]
        t = jnp.max(s3, axis=2)
        l = jnp.min(jnp.where(s3 == t[:, :, None], lane3, C), axis=2)
        vc_ref[0] = t
        cc_ref[0] = giota2 * C + l
        for k in range(1, NCACHE):
            keep = (s3 < t[:, :, None]) | (
                (s3 == t[:, :, None]) & (lane3 > l[:, :, None]))
            s3m = jnp.where(keep, s3, NEG)
            t = jnp.max(s3m, axis=2)
            l = jnp.min(jnp.where(s3m == t[:, :, None], lane3, C), axis=2)
            vc_ref[k] = t
            cc_ref[k] = giota2 * C + l

    giota2 = jax.lax.broadcasted_iota(jnp.int32, (R, NC), 1)
    v0 = vc_ref[0]
    m = jnp.max(v0, axis=1)
    gstar = jnp.min(jnp.where(v0 == m[:, None], giota2, NC), axis=1)
    onehot = giota2 == gstar[:, None]
    am = jnp.max(jnp.where(onehot, cc_ref[0], -1), axis=1)

    for k in range(NCACHE - 1):
        vc_ref[k] = jnp.where(onehot, vc_ref[k + 1], vc_ref[k])
        cc_ref[k] = jnp.where(onehot, cc_ref[k + 1], cc_ref[k])
    vc_ref[NCACHE - 1] = jnp.where(onehot, NEG, vc_ref[NCACHE - 1])

    head = jnp.max(jnp.where(onehot, vc_ref[0], NEG), axis=1)
    need = head == NEG

    @pl.when(jnp.any(need))
    def _():
        bias2 = jnp.where(need[:, None] & onehot, 0.0, NEG)
        sext = jnp.max(s3_ref[...] + bias2[:, :, None], axis=1)
        lane = jax.lax.broadcasted_iota(jnp.int32, (R, C), 1)
        colg = gstar[:, None] * C + lane
        mb = m[:, None]
        rem = jnp.where((sext < mb) | ((sext == mb) & (colg > am[:, None])),
                        sext, NEG)
        wmask = need[:, None] & onehot
        for k in range(NCACHE):
            t = jnp.max(rem, axis=1)
            c = jnp.min(jnp.where(rem == t[:, None], colg, BIG), axis=1)
            vc_ref[k] = jnp.where(wmask, t[:, None], vc_ref[k])
            cc_ref[k] = jnp.where(wmask, c[:, None], cc_ref[k])
            rem = jnp.where(colg == c[:, None], NEG, rem)

    krow = jax.lax.broadcasted_iota(jnp.int32, (K, 1, R), 0)
    sel_out = krow == j
    vals_ref[...] = jnp.where(sel_out, m[None, None, :], vals_ref[...])
    idx_ref[...] = jnp.where(sel_out, am[None, None, :], idx_ref[...])


@functools.partial(jax.jit)
def kernel(feature):
    nf = pl.pallas_call(
        _normalize_body,
        grid=(8,),
        in_specs=[pl.BlockSpec((N // 8, D), lambda i: (i, 0))],
        out_specs=pl.BlockSpec((N // 8, D), lambda i: (i, 0)),
        out_shape=jax.ShapeDtypeStruct((N, D), jnp.float32),
    )(feature)

    vals_t, idx_t = pl.pallas_call(
        _topk_body,
        grid=(NUM_BLOCKS, K),
        in_specs=[
            pl.BlockSpec(memory_space=pl.ANY),
        ],
        out_specs=[
            pl.BlockSpec((K, 1, R), lambda i, j: (0, 0, i)),
            pl.BlockSpec((K, 1, R), lambda i, j: (0, 0, i)),
        ],
        out_shape=[
            jax.ShapeDtypeStruct((K, 1, N), jnp.float32),
            jax.ShapeDtypeStruct((K, 1, N), jnp.int32),
        ],
        scratch_shapes=[
            pltpu.VMEM((R, NC, C), jnp.float32),
            pltpu.VMEM((NCACHE, R, NC), jnp.float32),
            pltpu.VMEM((NCACHE, R, NC), jnp.int32),
            pltpu.VMEM((N, D), jnp.float32),
            pltpu.SemaphoreType.DMA,
        ],
        compiler_params=pltpu.CompilerParams(
            dimension_semantics=("arbitrary", "arbitrary"),
        ),
    )(nf)

    src = jnp.tile(jnp.arange(N, dtype=jnp.int32), K)
    edge = jnp.stack([src, idx_t.reshape(-1)])
    edge_weights = vals_t.reshape(-1)
    return (edge, edge_weights)
